# baseline (speedup 1.0000x reference)
import functools
import math

import jax
import jax.numpy as jnp
from jax import lax
from jax.experimental import pallas as pl
from jax.experimental.pallas import tpu as pltpu
from jax.experimental.pallas import tpu_sc as plsc

_EMB_DIM = 128
_D_MODEL = 512
_N_ROWS = 50 * 1024

_NC = 2
_NS = 16
_NW = _NC * _NS
_CHUNK = 80
_RING = 6

_NSPLIT = 4
_BM = 6400

_L = 16


def _make_sc_body(ncch):
    rows_per_w = ncch * _CHUNK

    def convert_chunk(buf, obuf):
        obuf[...] = buf[...].astype(jnp.bfloat16)

    def body(table, idx_hbm, out_hbm, idx_v, *rest):
        bufs = rest[:_RING]
        obufs = rest[_RING:2 * _RING]
        gsems = rest[2 * _RING:3 * _RING]
        osems = rest[3 * _RING:4 * _RING]
        wid = lax.axis_index("s") * _NC + lax.axis_index("c")
        pltpu.sync_copy(idx_hbm.at[wid], idx_v)
        base = wid * rows_per_w
        g = [None] * _RING
        o = [None] * _RING
        for j in range(ncch + _RING - 1):
            if j < ncch:
                p = j % _RING
                g[p] = pltpu.async_copy(
                    table.at[idx_v.at[j]], bufs[p], gsems[p])
            d = j - (_RING - 1)
            if d >= 0:
                p = d % _RING
                g[p].wait()
                if o[p] is not None:
                    o[p].wait()
                convert_chunk(bufs[p], obufs[p])
                o[p] = pltpu.async_copy(
                    obufs[p], out_hbm.at[pl.ds(base + d * _CHUNK, _CHUNK)],
                    osems[p])
        for oc in o:
            if oc is not None:
                oc.wait()

    return body


def _sc_gather(table, idx3):
    nw, ncch, chunk = idx3.shape
    rows = nw * ncch * chunk
    mesh = plsc.VectorSubcoreMesh(
        core_axis_name="c", subcore_axis_name="s",
        num_cores=_NC, num_subcores=_NS)
    return pl.kernel(
        _make_sc_body(ncch),
        out_type=jax.ShapeDtypeStruct((rows, _EMB_DIM), jnp.bfloat16),
        mesh=mesh,
        scratch_types=(
            [pltpu.VMEM((ncch, chunk), jnp.int32)]
            + [pltpu.VMEM((chunk, _EMB_DIM), jnp.float32)] * _RING
            + [pltpu.VMEM((chunk, _EMB_DIM), jnp.bfloat16)] * _RING
            + [pltpu.SemaphoreType.DMA] * (2 * _RING)
        ),
    )(table, idx3)


def _mm_body(x_ref, w_ref, b_ref, o_ref):
    o_ref[...] = jnp.dot(
        x_ref[...], w_ref[...],
        preferred_element_type=jnp.float32) + b_ref[...]


def _mm_body_alias(x_ref, w_ref, b_ref, prev_ref, o_ref):
    del prev_ref
    o_ref[...] = jnp.dot(
        x_ref[...], w_ref[...],
        preferred_element_type=jnp.float32) + b_ref[...]


def _tc_matmul_chunk(x, w2, b2, prev, c, csteps):
    ins = [x, w2, b2]
    in_specs = [
        pl.BlockSpec((_BM, _EMB_DIM), lambda i: (i, 0)),
        pl.BlockSpec((_EMB_DIM, _D_MODEL), lambda i: (0, 0)),
        pl.BlockSpec((1, _D_MODEL), lambda i: (0, 0)),
    ]
    io_alias = {}
    body = _mm_body
    if prev is not None:
        ins.append(prev)
        in_specs.append(pl.BlockSpec(memory_space=pl.ANY))
        io_alias = {3: 0}
        body = _mm_body_alias
    return pl.pallas_call(
        body,
        grid=(csteps,),
        in_specs=in_specs,
        out_specs=pl.BlockSpec(
            (_BM, _D_MODEL), lambda i, c=c, s=csteps: (c * s + i, 0)),
        out_shape=jax.ShapeDtypeStruct((_N_ROWS, _D_MODEL), jnp.float32),
        input_output_aliases=io_alias,
        compiler_params=pltpu.CompilerParams(
            dimension_semantics=("parallel",)),
    )(*ins)


def kernel(persona, persona_pad_mask, emb_table, W, b):
    del persona_pad_mask
    seq, batch = persona.shape
    rows_c = _N_ROWS // _NSPLIT
    ncch = rows_c // _NW // _CHUNK
    csteps = rows_c // _BM
    idx = persona.reshape(_NSPLIT, _NW, ncch, _CHUNK)
    w2 = (W.T * jnp.float32(math.sqrt(_EMB_DIM))).astype(jnp.bfloat16)
    b2 = b.reshape(1, _D_MODEL)
    xs = [_sc_gather(emb_table, idx[c]) for c in range(_NSPLIT)]
    out = None
    for c in range(_NSPLIT):
        out = _tc_matmul_chunk(xs[c], w2, b2, out, c, csteps)
    return out.reshape(seq, batch, _D_MODEL)


# speedup vs baseline: 1.3608x; 1.3608x over previous
import functools
import math

import jax
import jax.numpy as jnp
from jax import lax
from jax.experimental import pallas as pl
from jax.experimental.pallas import tpu as pltpu
from jax.experimental.pallas import tpu_sc as plsc

_EMB_DIM = 128
_D_MODEL = 512
_N_ROWS = 50 * 1024

_NC = 2
_NS = 16
_NW = _NC * _NS
_CHUNK = 80
_RING = 6

_NSPLIT = 4
_BM = 6400

_L = 16


def _make_sc_body(ncch):
    rows_per_w = ncch * _CHUNK

    def convert_chunk(buf, obuf):
        def blk(i, _):
            s = pl.multiple_of(16 * i, 16)
            obuf[pl.ds(s, 16), :] = buf[pl.ds(s, 16), :].astype(
                jnp.bfloat16)
            return 0
        lax.fori_loop(0, _CHUNK // 16, blk, 0)

    def body(table, idx_hbm, out_hbm, idx_v, *rest):
        bufs = rest[:_RING]
        obufs = rest[_RING:2 * _RING]
        gsems = rest[2 * _RING:3 * _RING]
        osems = rest[3 * _RING:4 * _RING]
        wid = lax.axis_index("s") * _NC + lax.axis_index("c")
        pltpu.sync_copy(idx_hbm.at[wid], idx_v)
        base = wid * rows_per_w
        g = [None] * _RING
        o = [None] * _RING
        for j in range(ncch + _RING - 1):
            if j < ncch:
                p = j % _RING
                g[p] = pltpu.async_copy(
                    table.at[idx_v.at[j]], bufs[p], gsems[p])
            d = j - (_RING - 1)
            if d >= 0:
                p = d % _RING
                g[p].wait()
                if o[p] is not None:
                    o[p].wait()
                convert_chunk(bufs[p], obufs[p])
                o[p] = pltpu.async_copy(
                    obufs[p], out_hbm.at[pl.ds(base + d * _CHUNK, _CHUNK)],
                    osems[p])
        for oc in o:
            if oc is not None:
                oc.wait()

    return body


def _sc_gather(table, idx3):
    nw, ncch, chunk = idx3.shape
    rows = nw * ncch * chunk
    mesh = plsc.VectorSubcoreMesh(
        core_axis_name="c", subcore_axis_name="s",
        num_cores=_NC, num_subcores=_NS)
    return pl.kernel(
        _make_sc_body(ncch),
        out_type=jax.ShapeDtypeStruct((rows, _EMB_DIM), jnp.bfloat16),
        mesh=mesh,
        scratch_types=(
            [pltpu.VMEM((ncch, chunk), jnp.int32)]
            + [pltpu.VMEM((chunk, _EMB_DIM), jnp.float32)] * _RING
            + [pltpu.VMEM((chunk, _EMB_DIM), jnp.bfloat16)] * _RING
            + [pltpu.SemaphoreType.DMA] * (2 * _RING)
        ),
    )(table, idx3)


def _mm_body(x_ref, w_ref, b_ref, o_ref):
    o_ref[...] = jnp.dot(
        x_ref[...], w_ref[...],
        preferred_element_type=jnp.float32) + b_ref[...]


def _mm_body_alias(x_ref, w_ref, b_ref, prev_ref, o_ref):
    del prev_ref
    o_ref[...] = jnp.dot(
        x_ref[...], w_ref[...],
        preferred_element_type=jnp.float32) + b_ref[...]


def _tc_matmul_chunk(x, w2, b2, prev, c, csteps):
    ins = [x, w2, b2]
    in_specs = [
        pl.BlockSpec((_BM, _EMB_DIM), lambda i: (i, 0)),
        pl.BlockSpec((_EMB_DIM, _D_MODEL), lambda i: (0, 0)),
        pl.BlockSpec((1, _D_MODEL), lambda i: (0, 0)),
    ]
    io_alias = {}
    body = _mm_body
    if prev is not None:
        ins.append(prev)
        in_specs.append(pl.BlockSpec(memory_space=pl.ANY))
        io_alias = {3: 0}
        body = _mm_body_alias
    return pl.pallas_call(
        body,
        grid=(csteps,),
        in_specs=in_specs,
        out_specs=pl.BlockSpec(
            (_BM, _D_MODEL), lambda i, c=c, s=csteps: (c * s + i, 0)),
        out_shape=jax.ShapeDtypeStruct((_N_ROWS, _D_MODEL), jnp.float32),
        input_output_aliases=io_alias,
        compiler_params=pltpu.CompilerParams(
            dimension_semantics=("parallel",)),
    )(*ins)


def kernel(persona, persona_pad_mask, emb_table, W, b):
    del persona_pad_mask
    seq, batch = persona.shape
    rows_c = _N_ROWS // _NSPLIT
    ncch = rows_c // _NW // _CHUNK
    csteps = rows_c // _BM
    idx = persona.reshape(_NSPLIT, _NW, ncch, _CHUNK)
    w2 = (W.T * jnp.float32(math.sqrt(_EMB_DIM))).astype(jnp.bfloat16)
    b2 = b.reshape(1, _D_MODEL)
    xs = [_sc_gather(emb_table, idx[c]) for c in range(_NSPLIT)]
    out = None
    for c in range(_NSPLIT):
        out = _tc_matmul_chunk(xs[c], w2, b2, out, c, csteps)
    return out.reshape(seq, batch, _D_MODEL)


# speedup vs baseline: 1.7505x; 1.2864x over previous
import functools
import math

import jax
import jax.numpy as jnp
from jax import lax
from jax.experimental import pallas as pl
from jax.experimental.pallas import tpu as pltpu
from jax.experimental.pallas import tpu_sc as plsc

_EMB_DIM = 128
_D_MODEL = 512
_N_ROWS = 50 * 1024

_NC = 2
_NS = 16
_NW = _NC * _NS
_CHUNK = 80
_RING = 6

_NSPLIT = 1
_BM = 6400

_L = 16


def _make_sc_body(ncch):
    rows_per_w = ncch * _CHUNK

    def body(table, idx_hbm, out_hbm, idx_v, *rest):
        bufs = rest[:_RING]
        gsems = rest[_RING:2 * _RING]
        osems = rest[2 * _RING:3 * _RING]
        wid = lax.axis_index("s") * _NC + lax.axis_index("c")
        pltpu.sync_copy(idx_hbm.at[wid], idx_v)
        base = wid * rows_per_w
        g = [None] * _RING
        o = [None] * _RING
        for j in range(ncch + _RING - 1):
            if j < ncch:
                p = j % _RING
                if o[p] is not None:
                    o[p].wait()
                g[p] = pltpu.async_copy(
                    table.at[idx_v.at[j]], bufs[p], gsems[p])
            d = j - (_RING - 1)
            if d >= 0:
                p = d % _RING
                g[p].wait()
                o[p] = pltpu.async_copy(
                    bufs[p], out_hbm.at[pl.ds(base + d * _CHUNK, _CHUNK)],
                    osems[p])
        for oc in o:
            if oc is not None:
                oc.wait()

    return body


def _sc_gather(table, idx3):
    nw, ncch, chunk = idx3.shape
    rows = nw * ncch * chunk
    mesh = plsc.VectorSubcoreMesh(
        core_axis_name="c", subcore_axis_name="s",
        num_cores=_NC, num_subcores=_NS)
    return pl.kernel(
        _make_sc_body(ncch),
        out_type=jax.ShapeDtypeStruct((rows, _EMB_DIM), jnp.float32),
        mesh=mesh,
        scratch_types=(
            [pltpu.VMEM((ncch, chunk), jnp.int32)]
            + [pltpu.VMEM((chunk, _EMB_DIM), jnp.float32)] * _RING
            + [pltpu.SemaphoreType.DMA] * (2 * _RING)
        ),
    )(table, idx3)


def _mm_body(x_ref, w_ref, b_ref, o_ref):
    o_ref[...] = jnp.dot(
        x_ref[...], w_ref[...],
        preferred_element_type=jnp.float32) + b_ref[...]


def _mm_body_alias(x_ref, w_ref, b_ref, prev_ref, o_ref):
    del prev_ref
    o_ref[...] = jnp.dot(
        x_ref[...], w_ref[...],
        preferred_element_type=jnp.float32) + b_ref[...]


def _tc_matmul_chunk(x, w2, b2, prev, c, csteps):
    ins = [x, w2, b2]
    in_specs = [
        pl.BlockSpec((_BM, _EMB_DIM), lambda i: (i, 0)),
        pl.BlockSpec((_EMB_DIM, _D_MODEL), lambda i: (0, 0)),
        pl.BlockSpec((1, _D_MODEL), lambda i: (0, 0)),
    ]
    io_alias = {}
    body = _mm_body
    if prev is not None:
        ins.append(prev)
        in_specs.append(pl.BlockSpec(memory_space=pl.ANY))
        io_alias = {3: 0}
        body = _mm_body_alias
    return pl.pallas_call(
        body,
        grid=(csteps,),
        in_specs=in_specs,
        out_specs=pl.BlockSpec(
            (_BM, _D_MODEL), lambda i, c=c, s=csteps: (c * s + i, 0)),
        out_shape=jax.ShapeDtypeStruct((_N_ROWS, _D_MODEL), jnp.float32),
        input_output_aliases=io_alias,
        compiler_params=pltpu.CompilerParams(
            dimension_semantics=("parallel",)),
    )(*ins)


def kernel(persona, persona_pad_mask, emb_table, W, b):
    del persona_pad_mask
    seq, batch = persona.shape
    rows_c = _N_ROWS // _NSPLIT
    ncch = rows_c // _NW // _CHUNK
    csteps = rows_c // _BM
    idx = persona.reshape(_NSPLIT, _NW, ncch, _CHUNK)
    w2 = W.T * jnp.float32(math.sqrt(_EMB_DIM))
    b2 = b.reshape(1, _D_MODEL)
    xs = [_sc_gather(emb_table, idx[c]) for c in range(_NSPLIT)]
    out = None
    for c in range(_NSPLIT):
        out = _tc_matmul_chunk(xs[c], w2, b2, out, c, csteps)
    return out.reshape(seq, batch, _D_MODEL)
